# baseline (speedup 1.0000x reference)
import functools

import jax
import jax.numpy as jnp
from jax import lax
from jax.experimental import pallas as pl
from jax.experimental.pallas import tpu as pltpu
from jax.experimental.pallas import tpu_sc as plsc

S = 16384
NC = 1
NS = 16
L = 16
NW = NC * NS
ROWS = 128
COLS = S // ROWS


def _sc_kernel(n_pad):
    t = n_pad // NW
    groups = t // L

    @functools.partial(
        pl.kernel,
        out_type=jax.ShapeDtypeStruct((NC, ROWS, COLS), jnp.float32),
        mesh=plsc.VectorSubcoreMesh(
            core_axis_name="c", subcore_axis_name="s",
            num_cores=NC, num_subcores=NS),
        compiler_params=pltpu.CompilerParams(needs_layout_passes=False),
        scratch_types=[
            pltpu.VMEM((t,), jnp.float32),
            pltpu.VMEM((t,), jnp.float32),
            pltpu.VMEM((t,), jnp.float32),
            pltpu.VMEM((S,), jnp.float32),
            pltpu.VMEM((ROWS, COLS), jnp.float32),
            pltpu.VMEM((ROWS,), jnp.int32),
            pltpu.VMEM_SHARED((ROWS, COLS), jnp.float32),
        ],
    )
    def k(a_hbm, b_hbm, val_hbm, x_hbm, out_hbm, a_v, b_v, val_v, x_v, y_v,
          rows_v, y_shared):
        c = lax.axis_index("c")
        s = lax.axis_index("s")
        wid = c * NS + s
        base = wid * t

        zeros16 = jnp.zeros((L,), jnp.float32)
        iota16 = lax.iota(jnp.int32, L)

        def zero_body(i, _):
            y_v[i >> 3, pl.ds((i & 7) * L, L)] = zeros16
            return 0

        lax.fori_loop(0, ROWS * (COLS // L), zero_body, 0)

        def iota_body(i, _):
            rows_v[pl.ds(i * L, L)] = iota16 + i * L
            return 0

        lax.fori_loop(0, ROWS // L, iota_body, 0)

        @pl.when(s == 0)
        def _():
            pltpu.sync_copy(y_v, y_shared)

        pltpu.sync_copy(x_hbm, x_v)
        pltpu.sync_copy(a_hbm.at[pl.ds(base, t)], a_v)
        pltpu.sync_copy(b_hbm.at[pl.ds(base, t)], b_v)
        pltpu.sync_copy(val_hbm.at[pl.ds(base, t)], val_v)
        plsc.subcore_barrier()

        def body(g, _):
            sl = pl.ds(g * L, L)
            av = a_v[sl]
            bv = b_v[sl]
            v = val_v[sl]
            one16 = jnp.ones((L,), jnp.int32)
            zero16 = jnp.zeros((L,), jnp.int32)
            fai = av.astype(jnp.int32)
            fa = fai.astype(jnp.float32)
            cai = fai + jnp.where(av > fa, one16, zero16)
            ca = cai.astype(jnp.float32)
            fbi = bv.astype(jnp.int32)
            fb = fbi.astype(jnp.float32)
            cbi = fbi + jnp.where(bv > fb, one16, zero16)
            cb = cbi.astype(jnp.float32)

            xf = plsc.load_gather(x_v, [fbi])
            xc = plsc.load_gather(x_v, [cbi])

            g0 = (1.0 - (bv - fb)) * xf + (1.0 - (cb - bv)) * xc
            sf = v * (1.0 - (av - fa)) * g0
            sc = v * (1.0 - (ca - av)) * g0

            plsc.addupdate_scatter(y_v, [fai >> 7, fai & (COLS - 1)], sf)
            plsc.addupdate_scatter(y_v, [cai >> 7, cai & (COLS - 1)], sc)
            return 0

        lax.fori_loop(0, groups, body, 0)

        pltpu.sync_copy(y_v, y_shared.at[rows_v], add=True)
        plsc.subcore_barrier()

        @pl.when(s == 0)
        def _():
            pltpu.sync_copy(y_shared, out_hbm.at[c])

    return k


def _combine(p_ref, o_ref):
    o_ref[...] = p_ref[0] + p_ref[1]


def kernel(input, real_indices, real_values):
    n = real_indices.shape[0]
    n_pad = ((n + NW * L - 1) // (NW * L)) * (NW * L)
    pad = n_pad - n
    idx = jnp.pad(real_indices, ((0, pad), (0, 0)))
    val = jnp.pad(real_values, ((0, pad),))
    a_col = idx[:, 0]
    b_col = idx[:, 1]
    partials = _sc_kernel(n_pad)(a_col, b_col, val, input)
    if NC > 1:
        y = pl.pallas_call(
            _combine,
            out_shape=jax.ShapeDtypeStruct((ROWS, COLS), jnp.float32),
        )(partials)
    else:
        y = partials[0]
    return y.reshape(S)


# speedup vs baseline: 1.2991x; 1.2991x over previous
import functools

import jax
import jax.numpy as jnp
from jax import lax
from jax.experimental import pallas as pl
from jax.experimental.pallas import tpu as pltpu
from jax.experimental.pallas import tpu_sc as plsc

S = 16384
NC = 2
NS = 16
L = 16
NW = NC * NS
ROWS = 128
COLS = S // ROWS


def _sc_kernel(n_pad):
    t = n_pad // NW
    groups = t // L

    @functools.partial(
        pl.kernel,
        out_type=jax.ShapeDtypeStruct((NC, ROWS, COLS), jnp.float32),
        mesh=plsc.VectorSubcoreMesh(
            core_axis_name="c", subcore_axis_name="s",
            num_cores=NC, num_subcores=NS),
        compiler_params=pltpu.CompilerParams(needs_layout_passes=False),
        scratch_types=[
            pltpu.VMEM((t,), jnp.float32),
            pltpu.VMEM((t,), jnp.float32),
            pltpu.VMEM((t,), jnp.float32),
            pltpu.VMEM((S,), jnp.float32),
            pltpu.VMEM((ROWS, COLS), jnp.float32),
            pltpu.VMEM((ROWS,), jnp.int32),
            pltpu.VMEM_SHARED((ROWS, COLS), jnp.float32),
        ],
    )
    def k(a_hbm, b_hbm, val_hbm, x_hbm, out_hbm, a_v, b_v, val_v, x_v, y_v,
          rows_v, y_shared):
        c = lax.axis_index("c")
        s = lax.axis_index("s")
        wid = c * NS + s
        base = wid * t

        zeros16 = jnp.zeros((L,), jnp.float32)
        iota16 = lax.iota(jnp.int32, L)

        def zero_body(i, _):
            y_v[i >> 3, pl.ds((i & 7) * L, L)] = zeros16
            return 0

        lax.fori_loop(0, ROWS * (COLS // L), zero_body, 0)

        def iota_body(i, _):
            rows_v[pl.ds(i * L, L)] = iota16 + i * L
            return 0

        lax.fori_loop(0, ROWS // L, iota_body, 0)

        @pl.when(s == 0)
        def _():
            pltpu.sync_copy(y_v, y_shared)

        pltpu.sync_copy(x_hbm, x_v)
        pltpu.sync_copy(a_hbm.at[pl.ds(base, t)], a_v)
        pltpu.sync_copy(b_hbm.at[pl.ds(base, t)], b_v)
        pltpu.sync_copy(val_hbm.at[pl.ds(base, t)], val_v)
        plsc.subcore_barrier()

        one16 = jnp.ones((L,), jnp.int32)
        zero16 = jnp.zeros((L,), jnp.int32)
        fone16 = jnp.ones((L,), jnp.float32)

        @plsc.parallel_loop(0, groups, 1, unroll=4)
        def _(g):
            sl = pl.ds(g * L, L)
            av = a_v[sl]
            bv = b_v[sl]
            v = val_v[sl]
            fai = av.astype(jnp.int32)
            fa = fai.astype(jnp.float32)
            ta = av - fa
            ma = av > fa
            cai = fai + jnp.where(ma, one16, zero16)
            fbi = bv.astype(jnp.int32)
            fb = fbi.astype(jnp.float32)
            tb = bv - fb
            mb = bv > fb
            cbi = fbi + jnp.where(mb, one16, zero16)

            xf = plsc.load_gather(x_v, [fbi])
            xc = plsc.load_gather(x_v, [cbi])

            t0 = v * ((1.0 - tb) * xf + jnp.where(mb, tb, fone16) * xc)
            sf = (1.0 - ta) * t0
            sc = jnp.where(ma, ta, fone16) * t0

            plsc.addupdate_scatter(y_v, [fai >> 7, fai & (COLS - 1)], sf)
            plsc.addupdate_scatter(y_v, [cai >> 7, cai & (COLS - 1)], sc)

        pltpu.sync_copy(y_v, y_shared.at[rows_v], add=True)
        plsc.subcore_barrier()

        @pl.when(s == 0)
        def _():
            pltpu.sync_copy(y_shared, out_hbm.at[c])

    return k


def _combine(p_ref, o_ref):
    o_ref[...] = p_ref[0] + p_ref[1]


def kernel(input, real_indices, real_values):
    n = real_indices.shape[0]
    n_pad = ((n + NW * L - 1) // (NW * L)) * (NW * L)
    pad = n_pad - n
    idx = jnp.pad(real_indices, ((0, pad), (0, 0)))
    val = jnp.pad(real_values, ((0, pad),))
    a_col = idx[:, 0]
    b_col = idx[:, 1]
    partials = _sc_kernel(n_pad)(a_col, b_col, val, input)
    if NC > 1:
        y = pl.pallas_call(
            _combine,
            out_shape=jax.ShapeDtypeStruct((ROWS, COLS), jnp.float32),
        )(partials)
    else:
        y = partials[0]
    return y.reshape(S)
